# baseline (speedup 1.0000x reference)
import jax
import jax.numpy as jnp
from jax.experimental import pallas as pl
from jax.experimental.pallas import tpu as pltpu

D = 768
FF = 2048
E = 8
R = 8
ER = E * R
FB = 256


def _moe_kernel(x_ref, w1_ref, w3_ref, w2_ref, gw_ref, a1r_ref, a3r_ref,
                sm_ref, b2c_ref, out_ref, xbs, um, pp, msk, yac):
    j = pl.program_id(0)
    bf = jnp.bfloat16

    def fdot(a, b):
        return jax.lax.dot_general(
            a, b, (((1,), (1,)), ((), ())),
            preferred_element_type=jnp.float32)

    def bdot(a, b):
        return jax.lax.dot_general(
            a, b, (((1,), (0,)), ((), ())),
            preferred_element_type=jnp.float32)

    @pl.when(j == 0)
    def _prep():
        x = x_ref[...]
        logits = fdot(x, gw_ref[...])
        iota = jax.lax.broadcasted_iota(jnp.int32, logits.shape, 1)
        m1 = jnp.max(logits, axis=1, keepdims=True)
        i1 = jnp.min(jnp.where(logits == m1, iota, E), axis=1, keepdims=True)
        masked = jnp.where(iota == i1, -jnp.inf, logits)
        m2 = jnp.max(masked, axis=1, keepdims=True)
        i2 = jnp.min(jnp.where(masked == m2, iota, E), axis=1, keepdims=True)
        p1 = 1.0 / (1.0 + jnp.exp(m2 - m1))
        ecol = jax.lax.broadcasted_iota(jnp.int32, (x.shape[0], ER), 1) // R
        m1c = (ecol == i1).astype(jnp.float32)
        m2c = (ecol == i2).astype(jnp.float32)
        u1 = fdot(x, a1r_ref[...])
        u3 = fdot(x, a3r_ref[...])
        xbs[...] = x.astype(bf)
        um[:, 0 * ER:1 * ER] = (u1 * m1c).astype(bf)
        um[:, 1 * ER:2 * ER] = (u3 * m1c).astype(bf)
        um[:, 2 * ER:3 * ER] = (u1 * m2c).astype(bf)
        um[:, 3 * ER:4 * ER] = (u3 * m2c).astype(bf)
        pp[...] = jnp.broadcast_to(p1, pp.shape)
        msk[:, 0:ER] = m1c
        msk[:, ER:2 * ER] = m2c

    xb = xbs[...]
    w1b = w1_ref[...].astype(bf)
    w3b = w3_ref[...].astype(bf)
    w2b = w2_ref[...].astype(bf)
    c1 = fdot(xb, w1b)
    c3 = fdot(xb, w3b)
    b1c = sm_ref[0:ER, :]
    b3c = sm_ref[ER:2 * ER, :]
    a2c = sm_ref[2 * ER:3 * ER, :]

    p1 = pp[:, 0:1]
    acc = None
    ys = []
    for k in range(2):
        l1 = bdot(um[:, (2 * k) * ER:(2 * k + 1) * ER], b1c)
        l3 = bdot(um[:, (2 * k + 1) * ER:(2 * k + 2) * ER], b3c)
        w1e = c1 + l1
        w3e = c3 + l3
        p = p1 if k == 0 else 1.0 - p1
        s = (w1e * jax.nn.sigmoid(w1e)) * w3e * p
        ys.append(fdot(s.astype(bf), a2c))
        acc = s if acc is None else acc + s
    contrib = fdot(acc.astype(bf), w2b)

    @pl.when(j == 0)
    def _init():
        out_ref[...] = contrib
        yac[:, 0:ER] = ys[0]
        yac[:, ER:2 * ER] = ys[1]

    @pl.when(j > 0)
    def _accum():
        out_ref[...] += contrib
        yac[:, 0:ER] += ys[0]
        yac[:, ER:2 * ER] += ys[1]

    @pl.when(j == pl.num_programs(0) - 1)
    def _fin():
        y = (yac[:, 0:ER] * msk[:, 0:ER]
             + yac[:, ER:2 * ER] * msk[:, ER:2 * ER])
        out_ref[...] += bdot(y.astype(bf), b2c_ref[...])


def kernel(score_norm_data, W1, W3, W2, gate_W, A1, B1, A3, B3, A2, B2):
    T = score_norm_data.shape[0]
    B1c = jnp.swapaxes(B1, 1, 2).reshape(ER, FF)
    B3c = jnp.swapaxes(B3, 1, 2).reshape(ER, FF)
    A2c = A2.reshape(ER, FF)
    SM = jnp.concatenate([B1c, B3c, A2c], axis=0).astype(jnp.bfloat16)
    B2c = jnp.transpose(B2, (0, 2, 1)).reshape(ER, D).astype(jnp.bfloat16)
    A1r = A1.reshape(ER, D)
    A3r = A3.reshape(ER, D)

    full = lambda shape: pl.BlockSpec(shape, lambda j: (0,) * len(shape))
    return pl.pallas_call(
        _moe_kernel,
        grid=(FF // FB,),
        in_specs=[
            full((T, D)),
            pl.BlockSpec((FB, D), lambda j: (j, 0)),
            pl.BlockSpec((FB, D), lambda j: (j, 0)),
            pl.BlockSpec((D, FB), lambda j: (0, j)),
            full((E, D)),
            full((ER, D)), full((ER, D)),
            pl.BlockSpec((3 * ER, FB), lambda j: (0, j)),
            full((ER, D)),
        ],
        out_specs=full((T, D)),
        out_shape=jax.ShapeDtypeStruct((T, D), jnp.float32),
        scratch_shapes=[
            pltpu.VMEM((T, D), jnp.bfloat16),
            pltpu.VMEM((T, 4 * ER), jnp.bfloat16),
            pltpu.VMEM((T, 128), jnp.float32),
            pltpu.VMEM((T, 2 * ER), jnp.float32),
            pltpu.VMEM((T, 2 * ER), jnp.float32),
        ],
    )(score_norm_data, W1, W3, W2, gate_W, A1r, A3r, SM, B2c)


# speedup vs baseline: 1.1040x; 1.1040x over previous
import jax
import jax.numpy as jnp
from jax.experimental import pallas as pl
from jax.experimental.pallas import tpu as pltpu

D = 768
FF = 2048
E = 8
R = 8
ER = E * R
FB = 256


def _moe_kernel(x_ref, w1_ref, w3_ref, w2_ref, gw_ref, a1r_ref, a3r_ref,
                sm_ref, b2c_ref, out_ref, xbs, um, pp, msk, yac):
    j = pl.program_id(0)
    bf = jnp.bfloat16

    def fdot(a, b):
        return jax.lax.dot_general(
            a, b, (((1,), (1,)), ((), ())),
            preferred_element_type=jnp.float32)

    def bdot(a, b):
        return jax.lax.dot_general(
            a, b, (((1,), (0,)), ((), ())),
            preferred_element_type=jnp.float32)

    @pl.when(j == 0)
    def _prep():
        x = x_ref[...]
        logits = fdot(x, gw_ref[...])
        iota = jax.lax.broadcasted_iota(jnp.int32, logits.shape, 1)
        m1 = jnp.max(logits, axis=1, keepdims=True)
        i1 = jnp.min(jnp.where(logits == m1, iota, E), axis=1, keepdims=True)
        masked = jnp.where(iota == i1, -jnp.inf, logits)
        m2 = jnp.max(masked, axis=1, keepdims=True)
        i2 = jnp.min(jnp.where(masked == m2, iota, E), axis=1, keepdims=True)
        p1 = 1.0 / (1.0 + jnp.exp(m2 - m1))
        ecol = jax.lax.broadcasted_iota(jnp.int32, (x.shape[0], ER), 1) // R
        m1c = (ecol == i1).astype(jnp.float32)
        m2c = (ecol == i2).astype(jnp.float32)
        u1 = fdot(x, a1r_ref[...])
        u3 = fdot(x, a3r_ref[...])
        xbs[...] = x.astype(bf)
        um[:, 0 * ER:1 * ER] = (u1 * m1c).astype(bf)
        um[:, 1 * ER:2 * ER] = (u3 * m1c).astype(bf)
        um[:, 2 * ER:3 * ER] = (u1 * m2c).astype(bf)
        um[:, 3 * ER:4 * ER] = (u3 * m2c).astype(bf)
        pp[...] = jnp.broadcast_to(p1, pp.shape)
        msk[:, 0:ER] = m1c
        msk[:, ER:2 * ER] = m2c
        yac[...] = jnp.zeros_like(yac)
        out_ref[...] = jnp.zeros_like(out_ref)

    xb = xbs[...]
    w1b = w1_ref[...].astype(bf)
    w3b = w3_ref[...].astype(bf)
    w2b = w2_ref[...].astype(bf)
    c1 = fdot(xb, w1b)
    c3 = fdot(xb, w3b)
    b1c = sm_ref[0:ER, :]
    b3c = sm_ref[ER:2 * ER, :]
    a2c = sm_ref[2 * ER:3 * ER, :]

    p1 = pp[:, 0:1]
    acc = None
    for k in range(2):
        l1 = bdot(um[:, (2 * k) * ER:(2 * k + 1) * ER], b1c)
        l3 = bdot(um[:, (2 * k + 1) * ER:(2 * k + 2) * ER], b3c)
        w1e = c1 + l1
        w3e = c3 + l3
        p = p1 if k == 0 else 1.0 - p1
        s = (w1e * jax.nn.sigmoid(w1e)) * w3e * p
        yac[:, k * ER:(k + 1) * ER] += fdot(s.astype(bf), a2c)
        acc = s if acc is None else acc + s
    out_ref[...] += fdot(acc.astype(bf), w2b)

    @pl.when(j == pl.num_programs(0) - 1)
    def _fin():
        y = (yac[:, 0:ER] * msk[:, 0:ER]
             + yac[:, ER:2 * ER] * msk[:, ER:2 * ER])
        out_ref[...] += bdot(y.astype(bf), b2c_ref[...])


def kernel(score_norm_data, W1, W3, W2, gate_W, A1, B1, A3, B3, A2, B2):
    T = score_norm_data.shape[0]
    B1c = jnp.swapaxes(B1, 1, 2).reshape(ER, FF)
    B3c = jnp.swapaxes(B3, 1, 2).reshape(ER, FF)
    A2c = A2.reshape(ER, FF)
    SM = jnp.concatenate([B1c, B3c, A2c], axis=0).astype(jnp.bfloat16)
    B2c = jnp.transpose(B2, (0, 2, 1)).reshape(ER, D).astype(jnp.bfloat16)
    A1r = A1.reshape(ER, D)
    A3r = A3.reshape(ER, D)

    full = lambda shape: pl.BlockSpec(shape, lambda j: (0,) * len(shape))
    return pl.pallas_call(
        _moe_kernel,
        grid=(FF // FB,),
        in_specs=[
            full((T, D)),
            pl.BlockSpec((FB, D), lambda j: (j, 0)),
            pl.BlockSpec((FB, D), lambda j: (j, 0)),
            pl.BlockSpec((D, FB), lambda j: (0, j)),
            full((E, D)),
            full((ER, D)), full((ER, D)),
            pl.BlockSpec((3 * ER, FB), lambda j: (0, j)),
            full((ER, D)),
        ],
        out_specs=full((T, D)),
        out_shape=jax.ShapeDtypeStruct((T, D), jnp.float32),
        scratch_shapes=[
            pltpu.VMEM((T, D), jnp.bfloat16),
            pltpu.VMEM((T, 4 * ER), jnp.bfloat16),
            pltpu.VMEM((T, 128), jnp.float32),
            pltpu.VMEM((T, 2 * ER), jnp.float32),
            pltpu.VMEM((T, 2 * ER), jnp.float32),
        ],
    )(score_norm_data, W1, W3, W2, gate_W, A1r, A3r, SM, B2c)
